# baseline (speedup 1.0000x reference)
import functools

import jax
import jax.numpy as jnp
from jax import lax
from jax.experimental import pallas as pl
from jax.experimental.pallas import tpu as pltpu
from jax.experimental.pallas import tpu_sc as plsc

BATCH = 4
MAXLEN = 2048
DIM = 1024
NC = 2
NS = 16
NW = NC * NS

ROWS_PER_W = MAXLEN // NW
POS_WORDS = ROWS_PER_W * DIM
CHUNK_ROWS = 16
CHUNK_WORDS = CHUNK_ROWS * DIM
CHUNKS = ROWS_PER_W // CHUNK_ROWS
X_WORDS = BATCH * MAXLEN * DIM


def _sc_body(x_hbm, pos_hbm, out_hbm, pos_buf, x_buf):
    cid = lax.axis_index("c")
    sid = lax.axis_index("s")
    w = sid * NC + cid
    pbase = w * POS_WORDS
    pltpu.sync_copy(pos_hbm.at[pl.ds(pbase, POS_WORDS)], pos_buf)
    for b in range(BATCH):
        for c in range(CHUNKS):
            off = b * (MAXLEN * DIM) + pbase + c * CHUNK_WORDS

            def add_one(i, _, c=c):
                j = pl.multiple_of(i * 16, 16)
                x_buf[pl.ds(j, 16)] = (
                    x_buf[pl.ds(j, 16)] + pos_buf[pl.ds(c * CHUNK_WORDS + j, 16)]
                )
                return ()

            pltpu.sync_copy(x_hbm.at[pl.ds(off, CHUNK_WORDS)], x_buf)
            lax.fori_loop(0, CHUNK_WORDS // 16, add_one, ())
            pltpu.sync_copy(x_buf, out_hbm.at[pl.ds(off, CHUNK_WORDS)])


@jax.jit
def _pos_add(x, pos_table):
    mesh = plsc.VectorSubcoreMesh(core_axis_name="c", subcore_axis_name="s")
    run = pl.kernel(
        _sc_body,
        out_type=jax.ShapeDtypeStruct((X_WORDS,), jnp.float32),
        mesh=mesh,
        scratch_types=[
            pltpu.VMEM((POS_WORDS,), jnp.float32),
            pltpu.VMEM((CHUNK_WORDS,), jnp.float32),
        ],
    )
    return run(x.reshape(X_WORDS), pos_table.reshape(MAXLEN * DIM))


def kernel(x, pos_table):
    return _pos_add(x, pos_table).reshape(x.shape)


# speedup vs baseline: 1.7748x; 1.7748x over previous
import jax
import jax.numpy as jnp
from jax import lax
from jax.experimental import pallas as pl
from jax.experimental.pallas import tpu as pltpu
from jax.experimental.pallas import tpu_sc as plsc

BATCH = 4
MAXLEN = 2048
DIM = 1024
NC = 2
NS = 16
NW = NC * NS

ROWS_PER_W = MAXLEN // NW
POS_WORDS = ROWS_PER_W * DIM
CHUNK_ROWS = 16
CHUNK_WORDS = CHUNK_ROWS * DIM
PCHUNKS = ROWS_PER_W // CHUNK_ROWS
NCHUNKS = PCHUNKS * BATCH
XRING = 4
X_WORDS = BATCH * MAXLEN * DIM


def _sc_body(x_hbm, pos_hbm, out_hbm,
             p0, p1, x0, x1, x2, x3,
             ps0, ps1, is0, is1, is2, is3, os0, os1, os2, os3):
    pbufs = [p0, p1]
    xbufs = [x0, x1, x2, x3]
    psems = [ps0, ps1]
    isems = [is0, is1, is2, is3]
    osems = [os0, os1, os2, os3]

    cid = lax.axis_index("c")
    sid = lax.axis_index("s")
    w = sid * NC + cid
    pbase = w * POS_WORDS

    def x_off(k):
        c, b = divmod(k, BATCH)
        return b * (MAXLEN * DIM) + pbase + c * CHUNK_WORDS

    def start_pos(c):
        return pltpu.async_copy(
            pos_hbm.at[pl.ds(pbase + c * CHUNK_WORDS, CHUNK_WORDS)],
            pbufs[c % 2], psems[c % 2])

    def start_in(k):
        return pltpu.async_copy(
            x_hbm.at[pl.ds(x_off(k), CHUNK_WORDS)], xbufs[k % XRING],
            isems[k % XRING])

    def start_out(k):
        return pltpu.async_copy(
            xbufs[k % XRING], out_hbm.at[pl.ds(x_off(k), CHUNK_WORDS)],
            osems[k % XRING])

    pos_d = {0: start_pos(0)}
    in_d = {0: start_in(0), 1: start_in(1)}
    out_d = {}
    for k in range(NCHUNKS):
        c = k // BATCH
        if k % BATCH == 0:
            pos_d[c].wait()
            if c + 1 < PCHUNKS:
                pos_d[c + 1] = start_pos(c + 1)
        in_d[k].wait()
        xb = xbufs[k % XRING]
        pb = pbufs[c % 2]

        @plsc.parallel_loop(0, CHUNK_WORDS // 16, unroll=8)
        def add_one(i):
            j = pl.multiple_of(i * 16, 16)
            plsc.addupdate(xb.at[pl.ds(j, 16)], pb[pl.ds(j, 16)])

        out_d[k] = start_out(k)
        nxt = k + 2
        if nxt < NCHUNKS:
            if nxt - XRING >= 0:
                out_d[nxt - XRING].wait()
            in_d[nxt] = start_in(nxt)
    for k in range(NCHUNKS - XRING, NCHUNKS):
        out_d[k].wait()


@jax.jit
def _pos_add(x, pos_table):
    mesh = plsc.VectorSubcoreMesh(core_axis_name="c", subcore_axis_name="s")
    run = pl.kernel(
        _sc_body,
        out_type=jax.ShapeDtypeStruct((X_WORDS,), jnp.float32),
        mesh=mesh,
        scratch_types=(
            [pltpu.VMEM((CHUNK_WORDS,), jnp.float32)] * 2
            + [pltpu.VMEM((CHUNK_WORDS,), jnp.float32)] * XRING
            + [pltpu.SemaphoreType.DMA] * (2 + 2 * XRING)
        ),
    )
    return run(x.reshape(X_WORDS), pos_table.reshape(MAXLEN * DIM))


def kernel(x, pos_table):
    return _pos_add(x, pos_table).reshape(x.shape)


# speedup vs baseline: 4.3772x; 2.4662x over previous
import jax
import jax.numpy as jnp
from jax import lax
from jax.experimental import pallas as pl
from jax.experimental.pallas import tpu as pltpu
from jax.experimental.pallas import tpu_sc as plsc

BATCH = 4
MAXLEN = 2048
DIM = 1024
NC = 2
NS = 16
NW = NC * NS

ROWS_PER_W = MAXLEN // NW
CHUNK_ROWS = 16
PCHUNKS = ROWS_PER_W // CHUNK_ROWS
NCHUNKS = PCHUNKS * BATCH
XRING = 4
VECS = DIM // 16


def _sc_body(x_hbm, pos_hbm, out_hbm,
             p0, p1, x0, x1, x2, x3,
             ps0, ps1, is0, is1, is2, is3, os0, os1, os2, os3):
    pbufs = [p0, p1]
    xbufs = [x0, x1, x2, x3]
    psems = [ps0, ps1]
    isems = [is0, is1, is2, is3]
    osems = [os0, os1, os2, os3]

    cid = lax.axis_index("c")
    sid = lax.axis_index("s")
    w = sid * NC + cid
    prow = w * ROWS_PER_W

    def x_row(k):
        c, b = divmod(k, BATCH)
        return b * MAXLEN + prow + c * CHUNK_ROWS

    def start_pos(c):
        return pltpu.async_copy(
            pos_hbm.at[pl.ds(prow + c * CHUNK_ROWS, CHUNK_ROWS)],
            pbufs[c % 2], psems[c % 2])

    def start_in(k):
        return pltpu.async_copy(
            x_hbm.at[pl.ds(x_row(k), CHUNK_ROWS)], xbufs[k % XRING],
            isems[k % XRING])

    def start_out(k):
        return pltpu.async_copy(
            xbufs[k % XRING], out_hbm.at[pl.ds(x_row(k), CHUNK_ROWS)],
            osems[k % XRING])

    pos_d = {0: start_pos(0)}
    in_d = {0: start_in(0), 1: start_in(1)}
    out_d = {}
    for k in range(NCHUNKS):
        c = k // BATCH
        if k % BATCH == 0:
            pos_d[c].wait()
            if c + 1 < PCHUNKS:
                pos_d[c + 1] = start_pos(c + 1)
        in_d[k].wait()
        xb = xbufs[k % XRING]
        pb = pbufs[c % 2]

        @plsc.parallel_loop(0, CHUNK_ROWS * VECS, unroll=8)
        def add_one(i, xb=xb, pb=pb):
            r = i >> 6
            j = pl.multiple_of((i & (VECS - 1)) * 16, 16)
            plsc.addupdate(xb.at[r, pl.ds(j, 16)], pb[r, pl.ds(j, 16)])

        out_d[k] = start_out(k)
        nxt = k + 2
        if nxt < NCHUNKS:
            if nxt - XRING >= 0:
                out_d[nxt - XRING].wait()
            in_d[nxt] = start_in(nxt)
    for k in range(NCHUNKS - XRING, NCHUNKS):
        out_d[k].wait()


@jax.jit
def _pos_add(x, pos_table):
    mesh = plsc.VectorSubcoreMesh(core_axis_name="c", subcore_axis_name="s")
    run = pl.kernel(
        _sc_body,
        out_type=jax.ShapeDtypeStruct((BATCH * MAXLEN, DIM), jnp.float32),
        mesh=mesh,
        scratch_types=(
            [pltpu.VMEM((CHUNK_ROWS, DIM), jnp.float32)] * (2 + XRING)
            + [pltpu.SemaphoreType.DMA] * (2 + 2 * XRING)
        ),
    )
    return run(x.reshape(BATCH * MAXLEN, DIM), pos_table)


def kernel(x, pos_table):
    return _pos_add(x, pos_table).reshape(x.shape)


# speedup vs baseline: 4.4583x; 1.0185x over previous
import jax
import jax.numpy as jnp
from jax import lax
from jax.experimental import pallas as pl
from jax.experimental.pallas import tpu as pltpu
from jax.experimental.pallas import tpu_sc as plsc

BATCH = 4
MAXLEN = 2048
DIM = 1024
NC = 2
NS = 16
NW = NC * NS

ROWS_PER_W = MAXLEN // NW
CHUNK_ROWS = 16
PCHUNKS = ROWS_PER_W // CHUNK_ROWS
NCHUNKS = PCHUNKS * BATCH
XRING = 5
LOOKAHEAD = 3
VECS = DIM // 16


def _sc_body(x_hbm, pos_hbm, out_hbm,
             p0, p1, x0, x1, x2, x3, x4,
             ps0, ps1, is0, is1, is2, is3, is4, os0, os1, os2, os3, os4):
    pbufs = [p0, p1]
    xbufs = [x0, x1, x2, x3, x4]
    psems = [ps0, ps1]
    isems = [is0, is1, is2, is3, is4]
    osems = [os0, os1, os2, os3, os4]

    cid = lax.axis_index("c")
    sid = lax.axis_index("s")
    w = sid * NC + cid
    prow = w * ROWS_PER_W

    def x_row(k):
        c, b = divmod(k, BATCH)
        return b * MAXLEN + prow + c * CHUNK_ROWS

    def start_pos(c):
        return pltpu.async_copy(
            pos_hbm.at[pl.ds(prow + c * CHUNK_ROWS, CHUNK_ROWS)],
            pbufs[c % 2], psems[c % 2])

    def start_in(k):
        return pltpu.async_copy(
            x_hbm.at[pl.ds(x_row(k), CHUNK_ROWS)], xbufs[k % XRING],
            isems[k % XRING])

    def start_out(k):
        return pltpu.async_copy(
            xbufs[k % XRING], out_hbm.at[pl.ds(x_row(k), CHUNK_ROWS)],
            osems[k % XRING])

    pos_d = {0: start_pos(0)}
    in_d = {kk: start_in(kk) for kk in range(LOOKAHEAD)}
    out_d = {}
    for k in range(NCHUNKS):
        c = k // BATCH
        if k % BATCH == 0:
            pos_d[c].wait()
            if c + 1 < PCHUNKS:
                pos_d[c + 1] = start_pos(c + 1)
        in_d[k].wait()
        xb = xbufs[k % XRING]
        pb = pbufs[c % 2]

        @plsc.parallel_loop(0, CHUNK_ROWS * VECS, unroll=16)
        def add_one(i, xb=xb, pb=pb):
            r = i >> 6
            j = pl.multiple_of((i & (VECS - 1)) * 16, 16)
            plsc.addupdate(xb.at[r, pl.ds(j, 16)], pb[r, pl.ds(j, 16)])

        out_d[k] = start_out(k)
        nxt = k + LOOKAHEAD
        if nxt < NCHUNKS:
            if nxt - XRING >= 0:
                out_d[nxt - XRING].wait()
            in_d[nxt] = start_in(nxt)
    for k in range(NCHUNKS - XRING, NCHUNKS):
        out_d[k].wait()


@jax.jit
def _pos_add(x, pos_table):
    mesh = plsc.VectorSubcoreMesh(core_axis_name="c", subcore_axis_name="s")
    run = pl.kernel(
        _sc_body,
        out_type=jax.ShapeDtypeStruct((BATCH * MAXLEN, DIM), jnp.float32),
        mesh=mesh,
        scratch_types=(
            [pltpu.VMEM((CHUNK_ROWS, DIM), jnp.float32)] * (2 + XRING)
            + [pltpu.SemaphoreType.DMA] * (2 + 2 * XRING)
        ),
    )
    return run(x.reshape(BATCH * MAXLEN, DIM), pos_table)


def kernel(x, pos_table):
    return _pos_add(x, pos_table).reshape(x.shape)
